# baseline (speedup 1.0000x reference)
import functools

import jax
import jax.numpy as jnp
from jax import lax
from jax.experimental import pallas as pl
from jax.experimental.pallas import tpu as pltpu
from jax.experimental.pallas import tpu_sc as plsc

B, M, D, W = 64, 8192, 128, 33
WP = 40
PAD = WP - W
NWORKERS = 32
BPW = B // NWORKERS


def _copy_merge_body(idx_ref, src_ref, rows_ref, dst_ref):
    b = pl.program_id(0)
    dst_ref[...] = src_ref[...]
    for k in range(WP):
        r = idx_ref[b, k]
        dst_ref[0, pl.ds(r, 1), :] = rows_ref[0, pl.ds(k, 1), :]


_tc_copy_merge = pl.pallas_call(
    _copy_merge_body,
    grid_spec=pltpu.PrefetchScalarGridSpec(
        num_scalar_prefetch=1,
        grid=(B,),
        in_specs=[
            pl.BlockSpec((1, M, D), lambda b, idx_ref: (b, 0, 0)),
            pl.BlockSpec((1, WP, D), lambda b, idx_ref: (b, 0, 0)),
        ],
        out_specs=pl.BlockSpec((1, M, D), lambda b, idx_ref: (b, 0, 0)),
    ),
    out_shape=jax.ShapeDtypeStruct((B, M, D), jnp.float32),
)

_sc_mesh = plsc.VectorSubcoreMesh(core_axis_name="c", subcore_axis_name="s")


@functools.partial(
    pl.kernel,
    mesh=_sc_mesh,
    out_type=jax.ShapeDtypeStruct((B, WP, D), jnp.float32),
    scratch_types=[
        pltpu.VMEM((WP,), jnp.int32),
        pltpu.VMEM((WP, D), jnp.float32),
        pltpu.VMEM((WP, D), jnp.float32),
        pltpu.VMEM((WP, D), jnp.float32),
        pltpu.SemaphoreType.DMA,
    ],
)
def _sc_rows(ring, idxp, s1p, s2p, rows_out, idx_v, rows_v, s1_v, s2_v, sem):
    wid = lax.axis_index("s") * 2 + lax.axis_index("c")
    for j in range(BPW):
        b = wid * BPW + j
        pltpu.sync_copy(idxp.at[b], idx_v)
        pltpu.async_copy(ring.at[b].at[idx_v], rows_v, sem).wait()
        pltpu.sync_copy(s1p.at[b], s1_v)
        pltpu.sync_copy(s2p.at[b], s2_v)
        for r in range(WP):
            for c in range(D // 16):
                sl = (r, pl.ds(c * 16, 16))
                rows_v[sl] = rows_v[sl] * s1_v[sl] + s2_v[sl]
        pltpu.sync_copy(rows_v, rows_out.at[b])


def kernel(ring, write_vec, idx, weights, erase, write_gate):
    idx = idx.astype(jnp.int32)
    idxp = jnp.concatenate([jnp.broadcast_to(idx[:, -1:], (B, PAD)), idx], axis=1)
    wp = jnp.concatenate(
        [jnp.broadcast_to(weights[:, -1:], (B, PAD)), weights], axis=1)
    eq = idxp[:, :, None] == idxp[:, None, :]
    lastk = jnp.max(jnp.where(eq, jnp.arange(WP)[None, None, :], -1), axis=-1)
    s1 = 1.0 - erase[:, None] * wp
    s2 = write_gate[:, None] * wp
    s1d = jnp.take_along_axis(s1, lastk, axis=1)
    s2d = jnp.take_along_axis(s2, lastk, axis=1)
    s1p = jnp.broadcast_to(s1d[:, :, None], (B, WP, D))
    s2p = s2d[:, :, None] * write_vec[:, None, :]
    rows_upd = _sc_rows(ring, idxp, s1p, s2p)
    return _tc_copy_merge(idxp, ring, rows_upd)


# speedup vs baseline: 1.0666x; 1.0666x over previous
import functools

import jax
import jax.numpy as jnp
from jax import lax
from jax.experimental import pallas as pl
from jax.experimental.pallas import tpu as pltpu
from jax.experimental.pallas import tpu_sc as plsc

B, M, D, W = 64, 8192, 128, 33
WP = 40
PAD = WP - W
NWORKERS = 32
BPW = B // NWORKERS


def _copy_merge_body(idx_ref, src_ref, rows_ref, dst_ref):
    b = pl.program_id(0)
    dst_ref[...] = src_ref[...]
    for k in range(WP):
        r = idx_ref[b, k]
        dst_ref[0, pl.ds(r, 1), :] = rows_ref[0, pl.ds(k, 1), :]


_tc_copy_merge = pl.pallas_call(
    _copy_merge_body,
    grid_spec=pltpu.PrefetchScalarGridSpec(
        num_scalar_prefetch=1,
        grid=(B,),
        in_specs=[
            pl.BlockSpec((1, M, D), lambda b, idx_ref: (b, 0, 0)),
            pl.BlockSpec((1, WP, D), lambda b, idx_ref: (b, 0, 0)),
        ],
        out_specs=pl.BlockSpec((1, M, D), lambda b, idx_ref: (b, 0, 0)),
    ),
    out_shape=jax.ShapeDtypeStruct((B, M, D), jnp.float32),
    compiler_params=pltpu.CompilerParams(dimension_semantics=("parallel",)),
)

_sc_mesh = plsc.VectorSubcoreMesh(core_axis_name="c", subcore_axis_name="s")


@functools.partial(
    pl.kernel,
    mesh=_sc_mesh,
    out_type=jax.ShapeDtypeStruct((B, WP, D), jnp.float32),
    scratch_types=[
        pltpu.VMEM((WP,), jnp.int32),
        pltpu.VMEM((WP, D), jnp.float32),
        pltpu.VMEM((WP, D), jnp.float32),
        pltpu.VMEM((WP, D), jnp.float32),
        pltpu.SemaphoreType.DMA,
    ],
)
def _sc_rows(ring, idxp, s1p, s2p, rows_out, idx_v, rows_v, s1_v, s2_v, sem):
    wid = lax.axis_index("s") * 2 + lax.axis_index("c")
    for j in range(BPW):
        b = wid * BPW + j
        pltpu.sync_copy(idxp.at[b], idx_v)
        pltpu.async_copy(ring.at[b].at[idx_v], rows_v, sem).wait()
        pltpu.sync_copy(s1p.at[b], s1_v)
        pltpu.sync_copy(s2p.at[b], s2_v)
        for r in range(WP):
            for c in range(D // 16):
                sl = (r, pl.ds(c * 16, 16))
                rows_v[sl] = rows_v[sl] * s1_v[sl] + s2_v[sl]
        pltpu.sync_copy(rows_v, rows_out.at[b])


def kernel(ring, write_vec, idx, weights, erase, write_gate):
    idx = idx.astype(jnp.int32)
    idxp = jnp.concatenate([jnp.broadcast_to(idx[:, -1:], (B, PAD)), idx], axis=1)
    wp = jnp.concatenate(
        [jnp.broadcast_to(weights[:, -1:], (B, PAD)), weights], axis=1)
    s1 = 1.0 - erase[:, None] * wp
    s2 = write_gate[:, None] * wp
    s1p = jnp.broadcast_to(s1[:, :, None], (B, WP, D))
    s2p = s2[:, :, None] * write_vec[:, None, :]
    rows_upd = _sc_rows(ring, idxp, s1p, s2p)
    return _tc_copy_merge(idxp, ring, rows_upd)


# speedup vs baseline: 1.0754x; 1.0083x over previous
import functools

import jax
import jax.numpy as jnp
from jax import lax
from jax.experimental import pallas as pl
from jax.experimental.pallas import tpu as pltpu
from jax.experimental.pallas import tpu_sc as plsc

B, M, D, W = 64, 8192, 128, 33
WP = 40
PAD = WP - W
NWORKERS = 32
BPW = B // NWORKERS


BB = 2


def _copy_merge_body(idx_ref, src_ref, rows_ref, dst_ref):
    g = pl.program_id(0)
    dst_ref[...] = src_ref[...]
    for j in range(BB):
        for k in range(WP):
            r = idx_ref[g * BB + j, k]
            dst_ref[j, pl.ds(r, 1), :] = rows_ref[j, pl.ds(k, 1), :]


_tc_copy_merge = pl.pallas_call(
    _copy_merge_body,
    grid_spec=pltpu.PrefetchScalarGridSpec(
        num_scalar_prefetch=1,
        grid=(B // BB,),
        in_specs=[
            pl.BlockSpec((BB, M, D), lambda g, idx_ref: (g, 0, 0)),
            pl.BlockSpec((BB, WP, D), lambda g, idx_ref: (g, 0, 0)),
        ],
        out_specs=pl.BlockSpec((BB, M, D), lambda g, idx_ref: (g, 0, 0)),
    ),
    out_shape=jax.ShapeDtypeStruct((B, M, D), jnp.float32),
    compiler_params=pltpu.CompilerParams(dimension_semantics=("parallel",)),
)

_sc_mesh = plsc.VectorSubcoreMesh(core_axis_name="c", subcore_axis_name="s")


@functools.partial(
    pl.kernel,
    mesh=_sc_mesh,
    out_type=jax.ShapeDtypeStruct((B, WP, D), jnp.float32),
    scratch_types=[
        pltpu.VMEM((WP,), jnp.int32),
        pltpu.VMEM((WP, D), jnp.float32),
        pltpu.VMEM((WP, D), jnp.float32),
        pltpu.VMEM((WP, D), jnp.float32),
        pltpu.SemaphoreType.DMA,
    ],
)
def _sc_rows(ring, idxp, s1p, s2p, rows_out, idx_v, rows_v, s1_v, s2_v, sem):
    wid = lax.axis_index("s") * 2 + lax.axis_index("c")
    for j in range(BPW):
        b = wid * BPW + j
        pltpu.sync_copy(idxp.at[b], idx_v)
        pltpu.async_copy(ring.at[b].at[idx_v], rows_v, sem).wait()
        pltpu.sync_copy(s1p.at[b], s1_v)
        pltpu.sync_copy(s2p.at[b], s2_v)
        for r in range(WP):
            for c in range(D // 16):
                sl = (r, pl.ds(c * 16, 16))
                rows_v[sl] = rows_v[sl] * s1_v[sl] + s2_v[sl]
        pltpu.sync_copy(rows_v, rows_out.at[b])


def kernel(ring, write_vec, idx, weights, erase, write_gate):
    idx = idx.astype(jnp.int32)
    idxp = jnp.concatenate([jnp.broadcast_to(idx[:, -1:], (B, PAD)), idx], axis=1)
    wp = jnp.concatenate(
        [jnp.broadcast_to(weights[:, -1:], (B, PAD)), weights], axis=1)
    s1 = 1.0 - erase[:, None] * wp
    s2 = write_gate[:, None] * wp
    s1p = jnp.broadcast_to(s1[:, :, None], (B, WP, D))
    s2p = s2[:, :, None] * write_vec[:, None, :]
    rows_upd = _sc_rows(ring, idxp, s1p, s2p)
    return _tc_copy_merge(idxp, ring, rows_upd)


# speedup vs baseline: 1.1026x; 1.0253x over previous
import functools

import jax
import jax.numpy as jnp
from jax import lax
from jax.experimental import pallas as pl
from jax.experimental.pallas import tpu as pltpu
from jax.experimental.pallas import tpu_sc as plsc

B, M, D, W = 64, 8192, 128, 33
WP = 40
PAD = WP - W
NWORKERS = 32
BPW = B // NWORKERS


BB = 2
HB = B // 2
GB = HB // BB


def _make_copy_merge(off, aliased):
    def body(idx_ref, src_ref, rows_ref, *rest):
        dst_ref = rest[-1]
        g = pl.program_id(0)
        dst_ref[...] = src_ref[...]
        for j in range(BB):
            for k in range(WP):
                r = idx_ref[off + g * BB + j, k]
                dst_ref[j, pl.ds(r, 1), :] = rows_ref[j, pl.ds(k, 1), :]

    in_specs = [
        pl.BlockSpec((BB, M, D), lambda g, idx_ref: (g + off // BB, 0, 0)),
        pl.BlockSpec((BB, WP, D), lambda g, idx_ref: (g, 0, 0)),
    ]
    if aliased:
        in_specs.append(pl.BlockSpec(memory_space=pl.ANY))
    return pl.pallas_call(
        body,
        grid_spec=pltpu.PrefetchScalarGridSpec(
            num_scalar_prefetch=1,
            grid=(GB,),
            in_specs=in_specs,
            out_specs=pl.BlockSpec(
                (BB, M, D), lambda g, idx_ref: (g + off // BB, 0, 0)),
        ),
        out_shape=jax.ShapeDtypeStruct((B, M, D), jnp.float32),
        input_output_aliases={3: 0} if aliased else {},
        compiler_params=pltpu.CompilerParams(
            dimension_semantics=("parallel",)),
    )


_tc_copy_merge0 = _make_copy_merge(0, aliased=False)
_tc_copy_merge1 = _make_copy_merge(HB, aliased=True)

_sc_mesh = plsc.VectorSubcoreMesh(core_axis_name="c", subcore_axis_name="s")


def _make_sc_rows(off):
    @functools.partial(
        pl.kernel,
        mesh=_sc_mesh,
        out_type=jax.ShapeDtypeStruct((HB, WP, D), jnp.float32),
        scratch_types=[
            pltpu.VMEM((WP,), jnp.int32),
            pltpu.VMEM((WP, D), jnp.float32),
            pltpu.VMEM((WP, D), jnp.float32),
            pltpu.VMEM((WP, D), jnp.float32),
            pltpu.SemaphoreType.DMA,
        ],
    )
    def _sc_rows(ring, idxp, s1p, s2p, rows_out, idx_v, rows_v, s1_v, s2_v,
                 sem):
        bl = lax.axis_index("s") * 2 + lax.axis_index("c")
        b = off + bl
        pltpu.sync_copy(idxp.at[b], idx_v)
        pltpu.async_copy(ring.at[b].at[idx_v], rows_v, sem).wait()
        pltpu.sync_copy(s1p.at[b], s1_v)
        pltpu.sync_copy(s2p.at[b], s2_v)
        for r in range(WP):
            for c in range(D // 16):
                sl = (r, pl.ds(c * 16, 16))
                rows_v[sl] = rows_v[sl] * s1_v[sl] + s2_v[sl]
        pltpu.sync_copy(rows_v, rows_out.at[bl])

    return _sc_rows


_sc_rows0 = _make_sc_rows(0)
_sc_rows1 = _make_sc_rows(HB)


def kernel(ring, write_vec, idx, weights, erase, write_gate):
    idx = idx.astype(jnp.int32)
    idxp = jnp.concatenate([jnp.broadcast_to(idx[:, -1:], (B, PAD)), idx], axis=1)
    wp = jnp.concatenate(
        [jnp.broadcast_to(weights[:, -1:], (B, PAD)), weights], axis=1)
    s1 = 1.0 - erase[:, None] * wp
    s2 = write_gate[:, None] * wp
    s1p = jnp.broadcast_to(s1[:, :, None], (B, WP, D))
    s2p = s2[:, :, None] * write_vec[:, None, :]
    rows0 = _sc_rows0(ring, idxp, s1p, s2p)
    rows1 = _sc_rows1(ring, idxp, s1p, s2p)
    out = _tc_copy_merge0(idxp, ring, rows0)
    return _tc_copy_merge1(idxp, ring, rows1, out)
